# baseline (speedup 1.0000x reference)
import functools

import jax
import jax.numpy as jnp
from jax import lax
from jax.experimental import pallas as pl
from jax.experimental.pallas import tpu as pltpu
from jax.experimental.pallas import tpu_sc as plsc

_INFO = plsc.get_sparse_core_info()
_NC, _NS = _INFO.num_cores, _INFO.num_subcores
_NW = _NC * _NS

_CHUNK = 128
_NBUF = 2


def _emb_lookup(n_tokens: int, d: int):
    assert n_tokens % (_NW * _CHUNK * _NBUF) == 0
    per_w = n_tokens // _NW
    n_chunks = per_w // _CHUNK
    n_groups = n_chunks // _NBUF

    mesh = plsc.VectorSubcoreMesh(core_axis_name="c", subcore_axis_name="s")

    @functools.partial(
        pl.kernel,
        mesh=mesh,
        compiler_params=pltpu.CompilerParams(use_tc_tiling_on_sc=True),
        out_type=jax.ShapeDtypeStruct((n_tokens, d), jnp.float32),
        scratch_types=[
            pltpu.VMEM((n_chunks, _CHUNK), jnp.int32),
            pltpu.VMEM((_NBUF, _CHUNK, 128), jnp.float32),
            pltpu.VMEM((_NBUF, _CHUNK, d), jnp.float32),
        ]
        + [pltpu.SemaphoreType.DMA] * (2 * _NBUF),
    )
    def k(idx_hbm, table_hbm, out_hbm, idx_v, rows_v, packed_v, *sems):
        gsems, osems = sems[:_NBUF], sems[_NBUF:]
        wid = lax.axis_index("s") * _NC + lax.axis_index("c")
        base_chunk = wid * n_chunks
        pltpu.sync_copy(idx_hbm.at[pl.ds(base_chunk, n_chunks)], idx_v)

        def start_gather(j, b):
            pltpu.async_copy(table_hbm.at[idx_v.at[j]], rows_v.at[b], gsems[b])

        def wait_gather(b):
            pltpu.make_async_copy(
                table_hbm.at[pl.ds(0, _CHUNK)], rows_v.at[b], gsems[b]
            ).wait()

        def compact(b):
            def row_body(i, carry):
                t0 = i * 8
                for dt in range(8):
                    for c0 in range(0, d, 16):
                        packed_v[b, t0 + dt, pl.ds(c0, 16)] = rows_v[
                            b, t0 + dt, pl.ds(c0, 16)
                        ]
                return carry

            lax.fori_loop(0, _CHUNK // 8, row_body, 0)

        def start_out(j, b):
            row0 = (base_chunk + j) * _CHUNK
            pltpu.async_copy(packed_v.at[b], out_hbm.at[pl.ds(row0, _CHUNK)], osems[b])

        def wait_out(b):
            pltpu.make_async_copy(
                packed_v.at[b], out_hbm.at[pl.ds(0, _CHUNK)], osems[b]
            ).wait()

        for b in range(_NBUF):
            start_gather(b, b)

        def body(g, carry):
            j0 = g * _NBUF
            for b in range(_NBUF):
                wait_gather(b)
                compact(b)
                start_out(j0 + b, b)
            for b in range(_NBUF):
                nj = j0 + b + _NBUF

                @pl.when(nj < n_chunks)
                def _():
                    wait_out(b)
                    start_gather(nj, b)

            return carry

        lax.fori_loop(0, n_groups, body, 0)
        for b in range(_NBUF):
            wait_out(b)

    return k


def kernel(x, W):
    n_tokens = x.shape[0] * x.shape[1]
    d = W.shape[1]
    Wp = jnp.pad(W, ((0, 0), (0, 128 - d)))
    idx = x.astype(jnp.int32).reshape(n_tokens // _CHUNK, _CHUNK)
    out = _emb_lookup(n_tokens, d)(idx, Wp)
    return out.reshape(x.shape[0], x.shape[1], d)


# speedup vs baseline: 1.0243x; 1.0243x over previous
import functools

import jax
import jax.numpy as jnp
from jax import lax
from jax.experimental import pallas as pl
from jax.experimental.pallas import tpu as pltpu
from jax.experimental.pallas import tpu_sc as plsc

_INFO = plsc.get_sparse_core_info()
_NC, _NS = _INFO.num_cores, _INFO.num_subcores
_NW = _NC * _NS

_CHUNK = 64
_NBUF = 4


def _emb_lookup(n_tokens: int, d: int):
    assert n_tokens % (_NW * _CHUNK * _NBUF) == 0
    per_w = n_tokens // _NW
    n_chunks = per_w // _CHUNK
    n_groups = n_chunks // _NBUF

    mesh = plsc.VectorSubcoreMesh(core_axis_name="c", subcore_axis_name="s")

    @functools.partial(
        pl.kernel,
        mesh=mesh,
        compiler_params=pltpu.CompilerParams(use_tc_tiling_on_sc=True),
        out_type=jax.ShapeDtypeStruct((n_tokens, d), jnp.float32),
        scratch_types=[
            pltpu.VMEM((n_chunks, _CHUNK), jnp.int32),
            pltpu.VMEM((_NBUF, _CHUNK, 128), jnp.float32),
            pltpu.VMEM((_NBUF, _CHUNK, d), jnp.float32),
        ]
        + [pltpu.SemaphoreType.DMA] * (2 * _NBUF),
    )
    def k(idx_hbm, table_hbm, out_hbm, idx_v, rows_v, packed_v, *sems):
        gsems, osems = sems[:_NBUF], sems[_NBUF:]
        wid = lax.axis_index("s") * _NC + lax.axis_index("c")
        base_chunk = wid * n_chunks
        pltpu.sync_copy(idx_hbm.at[pl.ds(base_chunk, n_chunks)], idx_v)

        def start_gather(j, b):
            pltpu.async_copy(table_hbm.at[idx_v.at[j]], rows_v.at[b], gsems[b])

        def wait_gather(b):
            pltpu.make_async_copy(
                table_hbm.at[pl.ds(0, _CHUNK)], rows_v.at[b], gsems[b]
            ).wait()

        def compact(b):
            def row_body(i, carry):
                t0 = i * 8
                for dt in range(8):
                    for c0 in range(0, d, 16):
                        packed_v[b, t0 + dt, pl.ds(c0, 16)] = rows_v[
                            b, t0 + dt, pl.ds(c0, 16)
                        ]
                return carry

            lax.fori_loop(0, _CHUNK // 8, row_body, 0)

        def start_out(j, b):
            row0 = (base_chunk + j) * _CHUNK
            pltpu.async_copy(packed_v.at[b], out_hbm.at[pl.ds(row0, _CHUNK)], osems[b])

        def wait_out(b):
            pltpu.make_async_copy(
                packed_v.at[b], out_hbm.at[pl.ds(0, _CHUNK)], osems[b]
            ).wait()

        for b in range(_NBUF):
            start_gather(b, b)

        def body(g, carry):
            j0 = g * _NBUF
            for b in range(_NBUF):
                wait_gather(b)
                compact(b)
                start_out(j0 + b, b)
            for b in range(_NBUF):
                nj = j0 + b + _NBUF

                @pl.when(nj < n_chunks)
                def _():
                    wait_out(b)
                    start_gather(nj, b)

            return carry

        lax.fori_loop(0, n_groups, body, 0)
        for b in range(_NBUF):
            wait_out(b)

    return k


def kernel(x, W):
    n_tokens = x.shape[0] * x.shape[1]
    d = W.shape[1]
    Wp = jnp.pad(W, ((0, 0), (0, 128 - d)))
    idx = x.astype(jnp.int32).reshape(n_tokens // _CHUNK, _CHUNK)
    out = _emb_lookup(n_tokens, d)(idx, Wp)
    return out.reshape(x.shape[0], x.shape[1], d)
